# baseline (speedup 1.0000x reference)
import functools

import jax
import jax.numpy as jnp
from jax import lax
from jax.experimental import pallas as pl
from jax.experimental.pallas import tpu as pltpu
from jax.experimental.pallas import tpu_sc as plsc

B = 50000
N_NODES = 100000
N_NEIGHBORS = 16
NBR_DIM = 64
RAW_DIM = 128
MSG_DIM = 64

_INFO = plsc.get_sparse_core_info()
NC = _INFO.num_cores
NS = _INFO.num_subcores
HALF = N_NODES // NC
STAGE_PER_TILE = HALF // NS
N_STAGE = HALF + 8 * NS

E_PER_T = 3328
BPAD = NS * E_PER_T
CS = 208
N_CHUNKS = E_PER_T // CS
GRP = 208


def _sc_gather_sum(nbr_idx, table_bf16):
    mesh = plsc.VectorSubcoreMesh(core_axis_name="c", subcore_axis_name="s")

    @functools.partial(
        pl.kernel,
        out_type=jax.ShapeDtypeStruct((NC * BPAD, NBR_DIM), jnp.bfloat16),
        mesh=mesh,
        compiler_params=pltpu.CompilerParams(
            use_tc_tiling_on_sc=False, needs_layout_passes=False),
        scratch_types=[
            pltpu.VMEM_SHARED((N_STAGE, NBR_DIM), jnp.bfloat16),
            pltpu.VMEM((N_NEIGHBORS * CS,), jnp.int32),
            pltpu.VMEM((N_NEIGHBORS * CS,), jnp.int32),
            pltpu.VMEM((N_NEIGHBORS * CS,), jnp.int32),
            pltpu.VMEM((N_NEIGHBORS * CS,), jnp.int32),
            pltpu.VMEM((N_NEIGHBORS * CS,), jnp.int32),
            pltpu.VMEM((CS, NBR_DIM), jnp.bfloat16),
            pltpu.VMEM((CS, NBR_DIM), jnp.bfloat16),
            pltpu.VMEM((8, NBR_DIM), jnp.bfloat16),
            pltpu.SemaphoreType.DMA,
            pltpu.SemaphoreType.DMA,
            pltpu.SemaphoreType.DMA,
            pltpu.SemaphoreType.DMA,
            pltpu.SemaphoreType.DMA,
        ],
    )
    def body(nbr_hbm, table_hbm, out_hbm, stage_s, raw0_v, raw1_v,
             idx0_v, idx1_v, idx2_v, acc0_v, acc1_v, zrow_v,
             sem_i, sem_g, sem_a0, sem_a1, sem_o):
        cid = lax.axis_index("c")
        sid = lax.axis_index("s")
        lo = cid * HALF
        dummy = HALF + sid * 8
        obase = cid * BPAD + sid * E_PER_T

        pltpu.sync_copy(
            table_hbm.at[pl.ds(lo + sid * STAGE_PER_TILE, STAGE_PER_TILE)],
            stage_s.at[pl.ds(sid * STAGE_PER_TILE, STAGE_PER_TILE)],
        )
        for r in range(8):
            zrow_v[r, pl.ds(0, 32)] = jnp.zeros((32,), jnp.bfloat16)
            zrow_v[r, pl.ds(32, 32)] = jnp.zeros((32,), jnp.bfloat16)
        pltpu.sync_copy(zrow_v, stage_s.at[pl.ds(dummy, 8)])
        plsc.subcore_barrier()

        raws = (raw0_v, raw1_v)
        idxs = (idx0_v, idx1_v, idx2_v)
        accs = (acc0_v, acc1_v)
        sems = (sem_a0, sem_a1)
        lanes = lax.iota(jnp.int32, 16)
        fill = jnp.where(lanes == 0, dummy, -1)
        scat = lanes * CS

        def fire_idx(c):
            return pltpu.async_copy(
                nbr_hbm.at[pl.ds((sid * E_PER_T + c * CS) * N_NEIGHBORS,
                                 CS * N_NEIGHBORS)],
                raws[c % 2], sem_i)

        def localize(c):
            rv = raws[c % 2]
            iv = idxs[c % 3]

            def one(e, carry):
                raw = rv[pl.ds(e * 16, 16)]
                loc = raw - lo
                ok = (raw >= lo) & (loc < HALF)
                plsc.store_scatter(iv, [scat + e], jnp.where(ok, loc, fill))
                return carry

            lax.fori_loop(0, CS, one, 0)

        def fire_k0(c):
            iv = idxs[c % 3]
            return [
                pltpu.async_copy(
                    stage_s.at[iv.at[pl.ds(0, GRP)]],
                    accs[c % 2].at[pl.ds(0, GRP)],
                    sem_g,
                )
            ]

        def fire_adds(c):
            iv = idxs[c % 3]

            def add_round(k, carry):
                pltpu.async_copy(
                    stage_s.at[plsc.Indices(
                        iv.at[pl.ds(k * CS, CS)], ignored_value=-1)],
                    accs[c % 2],
                    sems[c % 2],
                    add=True,
                )
                return carry

            lax.fori_loop(1, N_NEIGHBORS, add_round, 0)

        def drain_adds(c):
            iv = idxs[c % 3]

            def drain_round(k, carry):
                pltpu.make_async_copy(
                    stage_s.at[plsc.Indices(
                        iv.at[pl.ds(k * CS, CS)], ignored_value=-1)],
                    accs[c % 2],
                    sems[c % 2],
                ).wait()
                return carry

            lax.fori_loop(1, N_NEIGHBORS, drain_round, 0)

        def fire_out(c):
            return pltpu.async_copy(
                accs[c % 2],
                out_hbm.at[pl.ds(obase + c * CS, CS)],
                sem_o,
            )

        fire_idx(0).wait()
        localize(0)
        k0_descs = fire_k0(0)
        idx_desc = fire_idx(1)
        for c in range(N_CHUNKS):
            for d in k0_descs:
                d.wait()
            fire_adds(c)
            if c > 0:
                drain_adds(c - 1)
                fire_out(c - 1).wait()
            if c + 2 < N_CHUNKS:
                next_idx_desc = fire_idx(c + 2)
            if c + 1 < N_CHUNKS:
                idx_desc.wait()
                localize(c + 1)
                k0_descs = fire_k0(c + 1)
                idx_desc = next_idx_desc if c + 2 < N_CHUNKS else None
        drain_adds(N_CHUNKS - 1)
        fire_out(N_CHUNKS - 1).wait()

    return body(nbr_idx, table_bf16)


def _msg_mlp_body(x_ref, w1_ref, b1_ref, w2_ref, b2_ref, a_ref):
    x = x_ref[...]
    h = jnp.maximum(
        jnp.dot(x, w1_ref[...], preferred_element_type=jnp.float32)
        + b1_ref[...], 0.0)
    a_ref[...] = jnp.maximum(
        jnp.dot(h, w2_ref[...], preferred_element_type=jnp.float32)
        + b2_ref[...], 0.0)


def _nbr_mlp_body(a_ref, p_ref, w3_ref, b3_ref, out_ref):
    p = p_ref[...].astype(jnp.float32)
    agg = (p[0] + p[1]) * (1.0 / N_NEIGHBORS)
    b_out = jnp.maximum(
        jnp.dot(agg, w3_ref[...], preferred_element_type=jnp.float32)
        + b3_ref[...], 0.0)
    out_ref[...] = a_ref[...] + b_out


def kernel(raw_messages, neighbors, memory_table, W1, b1, W2, b2, W3, b3):
    nbr = neighbors.astype(jnp.int32).reshape(-1)
    nbr = jnp.pad(nbr, (0, (BPAD - B) * N_NEIGHBORS))
    partials = _sc_gather_sum(nbr, memory_table.astype(jnp.bfloat16))
    partials = partials.reshape(NC, BPAD, NBR_DIM)

    blk = 2000
    grid = (B // blk,)
    a = pl.pallas_call(
        _msg_mlp_body,
        grid=grid,
        in_specs=[
            pl.BlockSpec((blk, RAW_DIM), lambda i: (i, 0)),
            pl.BlockSpec((RAW_DIM, RAW_DIM // 2), lambda i: (0, 0)),
            pl.BlockSpec((1, RAW_DIM // 2), lambda i: (0, 0)),
            pl.BlockSpec((RAW_DIM // 2, MSG_DIM), lambda i: (0, 0)),
            pl.BlockSpec((1, MSG_DIM), lambda i: (0, 0)),
        ],
        out_specs=pl.BlockSpec((blk, MSG_DIM), lambda i: (i, 0)),
        out_shape=jax.ShapeDtypeStruct((B, MSG_DIM), jnp.float32),
    )(raw_messages, W1, b1.reshape(1, -1), W2, b2.reshape(1, -1))
    out = pl.pallas_call(
        _nbr_mlp_body,
        grid=grid,
        in_specs=[
            pl.BlockSpec((blk, MSG_DIM), lambda i: (i, 0)),
            pl.BlockSpec((NC, blk, NBR_DIM), lambda i: (0, i, 0)),
            pl.BlockSpec((NBR_DIM, MSG_DIM), lambda i: (0, 0)),
            pl.BlockSpec((1, MSG_DIM), lambda i: (0, 0)),
        ],
        out_specs=pl.BlockSpec((blk, MSG_DIM), lambda i: (i, 0)),
        out_shape=jax.ShapeDtypeStruct((B, MSG_DIM), jnp.float32),
    )(a, partials, W3, b3.reshape(1, -1))
    return out


# speedup vs baseline: 1.0603x; 1.0603x over previous
import functools

import jax
import jax.numpy as jnp
from jax import lax
from jax.experimental import pallas as pl
from jax.experimental.pallas import tpu as pltpu
from jax.experimental.pallas import tpu_sc as plsc

B = 50000
N_NODES = 100000
N_NEIGHBORS = 16
NBR_DIM = 64
RAW_DIM = 128
MSG_DIM = 64

_INFO = plsc.get_sparse_core_info()
NC = _INFO.num_cores
NS = _INFO.num_subcores
HALF = N_NODES // NC
STAGE_PER_TILE = HALF // NS
N_STAGE = HALF + 8 * NS

E_PER_T = 3328
BPAD = NS * E_PER_T
CS = 256
N_CHUNKS = E_PER_T // CS
GRP = 256


def _sc_gather_sum(nbr_idx, table_bf16):
    mesh = plsc.VectorSubcoreMesh(core_axis_name="c", subcore_axis_name="s")

    @functools.partial(
        pl.kernel,
        out_type=jax.ShapeDtypeStruct((NC * BPAD, NBR_DIM), jnp.bfloat16),
        mesh=mesh,
        compiler_params=pltpu.CompilerParams(use_tc_tiling_on_sc=False),
        scratch_types=[
            pltpu.VMEM_SHARED((N_STAGE, NBR_DIM), jnp.bfloat16),
            pltpu.VMEM((N_NEIGHBORS, CS), jnp.int32),
            pltpu.VMEM((N_NEIGHBORS, CS), jnp.int32),
            pltpu.VMEM((N_NEIGHBORS, CS), jnp.int32),
            pltpu.VMEM((CS, NBR_DIM), jnp.bfloat16),
            pltpu.VMEM((CS, NBR_DIM), jnp.bfloat16),
            pltpu.VMEM((8, NBR_DIM), jnp.bfloat16),
            pltpu.SemaphoreType.DMA,
            pltpu.SemaphoreType.DMA,
            pltpu.SemaphoreType.DMA,
            pltpu.SemaphoreType.DMA,
            pltpu.SemaphoreType.DMA,
        ],
    )
    def body(nbr_hbm, table_hbm, out_hbm, stage_s, idx0_v, idx1_v, idx2_v,
             acc0_v, acc1_v, zrow_v, sem_i, sem_g, sem_a0, sem_a1, sem_o):
        cid = lax.axis_index("c")
        sid = lax.axis_index("s")
        lo = cid * HALF
        dummy = HALF + sid * 8
        obase = cid * BPAD + sid * E_PER_T

        pltpu.sync_copy(
            table_hbm.at[pl.ds(lo + sid * STAGE_PER_TILE, STAGE_PER_TILE)],
            stage_s.at[pl.ds(sid * STAGE_PER_TILE, STAGE_PER_TILE)],
        )
        for r in range(8):
            zrow_v[r, pl.ds(0, 32)] = jnp.zeros((32,), jnp.bfloat16)
            zrow_v[r, pl.ds(32, 32)] = jnp.zeros((32,), jnp.bfloat16)
        pltpu.sync_copy(zrow_v, stage_s.at[pl.ds(dummy, 8)])
        plsc.subcore_barrier()

        idxs = (idx0_v, idx1_v, idx2_v)
        accs = (acc0_v, acc1_v)
        sems = (sem_a0, sem_a1)

        def fire_idx(c):
            return pltpu.async_copy(nbr_hbm.at[sid, c], idxs[c % 3], sem_i)

        def localize(c):
            iv = idxs[c % 3]

            def one(i, carry):
                k = i // (CS // 16)
                j = (i % (CS // 16)) * 16
                raw = iv[k, pl.ds(j, 16)]
                loc = raw - lo
                ok = (raw >= lo) & (loc < HALF)
                fill = jnp.where(k == 0, dummy, -1)
                iv[k, pl.ds(j, 16)] = jnp.where(ok, loc, fill)
                return carry

            lax.fori_loop(0, N_NEIGHBORS * (CS // 16), one, 0)

        def fire_k0(c):
            iv = idxs[c % 3]
            return [
                pltpu.async_copy(
                    stage_s.at[iv.at[0, pl.ds(j * GRP, GRP)]],
                    accs[c % 2].at[pl.ds(j * GRP, GRP)],
                    sem_g,
                )
                for j in range(CS // GRP)
            ]

        def fire_adds(c):
            iv = idxs[c % 3]

            def add_round(k, carry):
                for j in range(CS // GRP):
                    pltpu.async_copy(
                        stage_s.at[plsc.Indices(
                            iv.at[k, pl.ds(j * GRP, GRP)], ignored_value=-1)],
                        accs[c % 2].at[pl.ds(j * GRP, GRP)],
                        sems[c % 2],
                        add=True,
                    )
                return carry

            lax.fori_loop(1, N_NEIGHBORS, add_round, 0)

        def drain_adds(c):
            iv = idxs[c % 3]

            def drain_round(k, carry):
                for j in range(CS // GRP):
                    pltpu.make_async_copy(
                        stage_s.at[plsc.Indices(
                            iv.at[k, pl.ds(j * GRP, GRP)], ignored_value=-1)],
                        accs[c % 2].at[pl.ds(j * GRP, GRP)],
                        sems[c % 2],
                    ).wait()
                return carry

            lax.fori_loop(1, N_NEIGHBORS, drain_round, 0)

        def fire_out(c):
            return pltpu.async_copy(
                accs[c % 2],
                out_hbm.at[pl.ds(obase + c * CS, CS)],
                sem_o,
            )

        fire_idx(0).wait()
        localize(0)
        k0_descs = fire_k0(0)
        idx_desc = fire_idx(1)
        for c in range(N_CHUNKS):
            for d in k0_descs:
                d.wait()
            fire_adds(c)
            if c > 0:
                drain_adds(c - 1)
                fire_out(c - 1).wait()
            if c + 2 < N_CHUNKS:
                next_idx_desc = fire_idx(c + 2)
            if c + 1 < N_CHUNKS:
                idx_desc.wait()
                localize(c + 1)
                k0_descs = fire_k0(c + 1)
                idx_desc = next_idx_desc if c + 2 < N_CHUNKS else None
        drain_adds(N_CHUNKS - 1)
        fire_out(N_CHUNKS - 1).wait()

    return body(nbr_idx, table_bf16)


def _msg_mlp_body(x_ref, w1_ref, b1_ref, w2_ref, b2_ref, a_ref):
    x = x_ref[...]
    h = jnp.maximum(
        jnp.dot(x, w1_ref[...], preferred_element_type=jnp.float32)
        + b1_ref[...], 0.0)
    a_ref[...] = jnp.maximum(
        jnp.dot(h, w2_ref[...], preferred_element_type=jnp.float32)
        + b2_ref[...], 0.0)


def _nbr_mlp_body(a_ref, p_ref, w3_ref, b3_ref, out_ref):
    p = p_ref[...].astype(jnp.float32)
    agg = (p[0] + p[1]) * (1.0 / N_NEIGHBORS)
    b_out = jnp.maximum(
        jnp.dot(agg, w3_ref[...], preferred_element_type=jnp.float32)
        + b3_ref[...], 0.0)
    out_ref[...] = a_ref[...] + b_out


def kernel(raw_messages, neighbors, memory_table, W1, b1, W2, b2, W3, b3):
    nbr = neighbors.astype(jnp.int32)
    nbr = jnp.pad(nbr, ((0, BPAD - B), (0, 0)))
    nbr = nbr.reshape(NS, N_CHUNKS, CS, N_NEIGHBORS)
    nbr = nbr.transpose(0, 1, 3, 2)
    partials = _sc_gather_sum(nbr, memory_table.astype(jnp.bfloat16))
    partials = partials.reshape(NC, BPAD, NBR_DIM)

    blk = 2000
    grid = (B // blk,)
    a = pl.pallas_call(
        _msg_mlp_body,
        grid=grid,
        in_specs=[
            pl.BlockSpec((blk, RAW_DIM), lambda i: (i, 0)),
            pl.BlockSpec((RAW_DIM, RAW_DIM // 2), lambda i: (0, 0)),
            pl.BlockSpec((1, RAW_DIM // 2), lambda i: (0, 0)),
            pl.BlockSpec((RAW_DIM // 2, MSG_DIM), lambda i: (0, 0)),
            pl.BlockSpec((1, MSG_DIM), lambda i: (0, 0)),
        ],
        out_specs=pl.BlockSpec((blk, MSG_DIM), lambda i: (i, 0)),
        out_shape=jax.ShapeDtypeStruct((B, MSG_DIM), jnp.float32),
    )(raw_messages, W1, b1.reshape(1, -1), W2, b2.reshape(1, -1))
    out = pl.pallas_call(
        _nbr_mlp_body,
        grid=grid,
        in_specs=[
            pl.BlockSpec((blk, MSG_DIM), lambda i: (i, 0)),
            pl.BlockSpec((NC, blk, NBR_DIM), lambda i: (0, i, 0)),
            pl.BlockSpec((NBR_DIM, MSG_DIM), lambda i: (0, 0)),
            pl.BlockSpec((1, MSG_DIM), lambda i: (0, 0)),
        ],
        out_specs=pl.BlockSpec((blk, MSG_DIM), lambda i: (i, 0)),
        out_shape=jax.ShapeDtypeStruct((B, MSG_DIM), jnp.float32),
    )(a, partials, W3, b3.reshape(1, -1))
    return out
